# baseline (speedup 1.0000x reference)
import functools

import jax
import jax.numpy as jnp
import numpy as np
from jax import lax
from jax.experimental import pallas as pl
from jax.experimental.pallas import tpu as pltpu
from jax.experimental.pallas import tpu_sc as plsc

_B, _S, _C = 4, 4096, 1024
_K = 64
_L = 16
_CB = 128
_NSUB = _CB // _L
_NBINS = 256
_NC, _NS = 2, 16
_R = 128
_NCHUNK = _S // _R
_SAMPLE = 4
_SNEED = 25
_CAND = 384

_TOPBIT = np.uint32(0x80000000)


def _key_of(v):
  i = lax.bitcast_convert_type(v, jnp.int32)
  m = lax.bitcast_convert_type(i >> jnp.int32(31), jnp.uint32) | _TOPBIT
  return lax.bitcast_convert_type(i, jnp.uint32) ^ m


def _val_of(key):
  u = jnp.where(key >= _TOPBIT, key ^ _TOPBIT, ~key)
  return lax.bitcast_convert_type(u, jnp.float32)


def _kmax_body(x_hbm, out_hbm, dbuf0, dbuf1, hist, outb, clv, sem0, sem1):
  cid = lax.axis_index("c")
  sid = lax.axis_index("s")
  wid = sid * _NC + cid
  b = wid // 8
  c0 = (wid % 8) * _CB
  lanes = lax.iota(jnp.int32, _L)
  ones = jnp.ones((_L,), jnp.int32)
  zi = jnp.zeros((_L,), jnp.int32)

  def slab(ci):
    return x_hbm.at[b, pl.ds(ci * _R, _R), pl.ds(c0, _CB)]

  def clear_hist():
    def clr(j, _):
      for s in range(_NSUB):
        hist[j, pl.ds(s * _L, _L)] = zi
      return 0
    lax.fori_loop(0, _NBINS, clr, 0)

  def stream_pass(row_fn, carry, step=1):
    def process(buf, carry):
      def row_body(i, carry):
        vecs = [buf[i * step, pl.ds(s * _L, _L)] for s in range(_NSUB)]
        return row_fn(vecs, carry)
      return lax.fori_loop(0, _R // step, row_body, carry)

    def pair_body(j, carry):
      pltpu.async_copy(slab(2 * j + 1), dbuf1, sem1)
      pltpu.make_async_copy(slab(0), dbuf0, sem0).wait()
      carry = process(dbuf0, carry)
      nxt = (2 * j + 2) % _NCHUNK
      pltpu.async_copy(slab(nxt), dbuf0, sem0)
      pltpu.make_async_copy(slab(0), dbuf1, sem1).wait()
      carry = process(dbuf1, carry)
      return carry
    return lax.fori_loop(0, _NCHUNK // 2, pair_body, carry)

  def scan_hist(s, need):
    def sc(j, carry):
      cum, p, above, found = carry
      bin_j = _NBINS - 1 - j
      cnt = hist[bin_j, pl.ds(s * _L, _L)]
      newcum = cum + cnt
      cross = jnp.logical_and(jnp.logical_not(found), newcum >= need)
      p = jnp.where(cross, bin_j, p)
      above = jnp.where(cross, cum, above)
      found = jnp.logical_or(found, cross)
      return newcum, p, above, found
    init = (zi, zi, zi, jnp.zeros((_L,), jnp.bool_))
    _, p, above, _ = lax.fori_loop(0, _NBINS, sc, init)
    return p, above

  pltpu.async_copy(slab(0), dbuf0, sem0)

  clear_hist()

  def hist1_row(vecs, carry):
    for s in range(_NSUB):
      key = _key_of(vecs[s])
      bn = lax.bitcast_convert_type(key >> jnp.uint32(24), jnp.int32)
      plsc.addupdate_scatter(hist, [bn, lanes + s * _L], ones)
    return carry
  stream_pass(hist1_row, 0, step=_SAMPLE)

  phats = []
  ok = jnp.bool_(True)
  for s in range(_NSUB):
    p, _ = scan_hist(s, jnp.full((_L,), _SNEED, jnp.int32))
    phats.append(p)
    ok = jnp.logical_and(ok, jnp.all(p >= 129))

  cutoffs = [lax.bitcast_convert_type((phats[s] - 128) << jnp.int32(24),
                                      jnp.float32)
             for s in range(_NSUB)]

  def collect_row(vecs, carry):
    ccnts = list(carry)
    for s in range(_NSUB):
      m = vecs[s] >= cutoffs[s]
      mw = jnp.logical_and(m, ccnts[s] < _CAND)
      plsc.store_scatter(clv, [ccnts[s], lanes + s * _L], vecs[s], mask=mw)
      ccnts[s] = ccnts[s] + jnp.where(m, 1, 0)
    return tuple(ccnts)
  ccnts = list(stream_pass(collect_row, tuple([zi] * _NSUB)))

  maxcc = jnp.int32(0)
  for s in range(_NSUB):
    ok = jnp.logical_and(ok, jnp.all(ccnts[s] >= _K))
    ok = jnp.logical_and(ok, jnp.all(ccnts[s] <= _CAND))
    maxcc = jnp.maximum(maxcc, jnp.max(ccnts[s]))

  def fast_path():
    prefixes = [jnp.zeros((_L,), jnp.uint32) for _ in range(_NSUB)]
    needs = [jnp.full((_L,), _K, jnp.int32) for _ in range(_NSUB)]
    for shift in (24, 16, 8, 0):
      clear_hist()

      def cr(j, _, prefixes=prefixes, shift=shift):
        for s in range(_NSUB):
          kj = _key_of(clv[j, pl.ds(s * _L, _L)])
          m = j < ccnts[s]
          if shift < 24:
            m = jnp.logical_and(
                m, (kj >> jnp.uint32(shift + 8)) == prefixes[s])
          bn = lax.bitcast_convert_type(
              (kj >> jnp.uint32(shift)) & jnp.uint32(0xFF), jnp.int32)
          plsc.addupdate_scatter(hist, [bn, lanes + s * _L], ones, mask=m)
        return 0
      lax.fori_loop(0, maxcc, cr, 0)

      for s in range(_NSUB):
        p, above = scan_hist(s, needs[s])
        prefixes[s] = (
            (prefixes[s] << jnp.uint32(8))
            | lax.bitcast_convert_type(p, jnp.uint32))
        needs[s] = needs[s] - above

    tvals = [_val_of(prefixes[s]) for s in range(_NSUB)]

    def emit(j, carry):
      outcnts, ties = list(carry[0]), list(carry[1])
      for s in range(_NSUB):
        vj = clv[j, pl.ds(s * _L, _L)]
        valid = j < ccnts[s]
        gt = vj > tvals[s]
        eq = jnp.logical_and(vj == tvals[s], ties[s] < needs[s])
        take = jnp.logical_and(valid, jnp.logical_or(gt, eq))
        plsc.store_scatter(outb, [outcnts[s], lanes + s * _L], vj,
                           mask=take)
        outcnts[s] = outcnts[s] + jnp.where(take, 1, 0)
        ties[s] = ties[s] + jnp.where(jnp.logical_and(take, eq), 1, 0)
      return tuple(outcnts), tuple(ties)
    lax.fori_loop(0, maxcc, emit, (tuple([zi] * _NSUB),
                                   tuple([zi] * _NSUB)))
    return 0

  def slow_path():
    prefixes = [jnp.zeros((_L,), jnp.uint32) for _ in range(_NSUB)]
    needs = [jnp.full((_L,), _K, jnp.int32) for _ in range(_NSUB)]
    for shift in (24, 16, 8, 0):
      clear_hist()

      def histk_row(vecs, carry, prefixes=prefixes, shift=shift):
        for s in range(_NSUB):
          key = _key_of(vecs[s])
          bn = lax.bitcast_convert_type(
              (key >> jnp.uint32(shift)) & jnp.uint32(0xFF), jnp.int32)
          if shift == 24:
            plsc.addupdate_scatter(hist, [bn, lanes + s * _L], ones)
          else:
            m = (key >> jnp.uint32(shift + 8)) == prefixes[s]
            plsc.addupdate_scatter(hist, [bn, lanes + s * _L], ones,
                                   mask=m)
        return carry
      stream_pass(histk_row, 0)

      for s in range(_NSUB):
        p, above = scan_hist(s, needs[s])
        prefixes[s] = (
            (prefixes[s] << jnp.uint32(8))
            | lax.bitcast_convert_type(p, jnp.uint32))
        needs[s] = needs[s] - above

    def sel_row(vecs, carry):
      outcnts, ties = list(carry[0]), list(carry[1])
      for s in range(_NSUB):
        key = _key_of(vecs[s])
        gt = key > prefixes[s]
        take_eq = jnp.logical_and(key == prefixes[s], ties[s] < needs[s])
        take = jnp.logical_or(gt, take_eq)
        plsc.store_scatter(outb, [outcnts[s], lanes + s * _L], vecs[s],
                           mask=take)
        outcnts[s] = outcnts[s] + jnp.where(take, 1, 0)
        ties[s] = ties[s] + jnp.where(take_eq, 1, 0)
      return tuple(outcnts), tuple(ties)
    stream_pass(sel_row, (tuple([zi] * _NSUB), tuple([zi] * _NSUB)))
    return 0

  lax.cond(ok, fast_path, slow_path)

  pltpu.make_async_copy(slab(0), dbuf0, sem0).wait()

  pltpu.sync_copy(outb, out_hbm.at[b, :, pl.ds(c0, _CB)])


@functools.partial(
    pl.kernel,
    out_type=jax.ShapeDtypeStruct((_B, _K, _C), jnp.float32),
    mesh=plsc.VectorSubcoreMesh(
        core_axis_name="c", subcore_axis_name="s",
        num_cores=_NC, num_subcores=_NS),
    scratch_types=[
        pltpu.VMEM((_R, _CB), jnp.float32),
        pltpu.VMEM((_R, _CB), jnp.float32),
        pltpu.VMEM((_NBINS, _CB), jnp.int32),
        pltpu.VMEM((_K, _CB), jnp.float32),
        pltpu.VMEM((_CAND, _CB), jnp.float32),
        pltpu.SemaphoreType.DMA,
        pltpu.SemaphoreType.DMA,
    ],
    compiler_params=pltpu.CompilerParams(needs_layout_passes=False),
)
def _kmax_sc(x_hbm, out_hbm, dbuf0, dbuf1, hist, outb, clv, sem0, sem1):
  _kmax_body(x_hbm, out_hbm, dbuf0, dbuf1, hist, outb, clv, sem0, sem1)


def kernel(x):
  return _kmax_sc(x)


# speedup vs baseline: 2.4698x; 2.4698x over previous
import functools

import jax
import jax.numpy as jnp
import numpy as np
from jax import lax
from jax.experimental import pallas as pl
from jax.experimental.pallas import tpu as pltpu
from jax.experimental.pallas import tpu_sc as plsc

_B, _S, _C = 4, 4096, 1024
_K = 64
_L = 16
_CB = 128
_NSUB = _CB // _L
_NBINS = 256
_NC, _NS = 2, 16
_R = 128
_NCHUNK = _S // _R
_CAND = 384

_TOPBIT = np.uint32(0x80000000)


def _key_of(v):
  i = lax.bitcast_convert_type(v, jnp.int32)
  m = lax.bitcast_convert_type(i >> jnp.int32(31), jnp.uint32) | _TOPBIT
  return lax.bitcast_convert_type(i, jnp.uint32) ^ m


def _val_of(key):
  u = jnp.where(key >= _TOPBIT, key ^ _TOPBIT, ~key)
  return lax.bitcast_convert_type(u, jnp.float32)


def _kmax_body(x_hbm, out_hbm, dbuf0, dbuf1, hist, outb, clk, sem0, sem1):
  cid = lax.axis_index("c")
  sid = lax.axis_index("s")
  wid = sid * _NC + cid
  b = wid // 8
  c0 = (wid % 8) * _CB
  lanes = lax.iota(jnp.int32, _L)
  ones = jnp.ones((_L,), jnp.int32)
  zi = jnp.zeros((_L,), jnp.int32)

  def slab(ci):
    return x_hbm.at[b, pl.ds(ci * _R, _R), pl.ds(c0, _CB)]

  def clear_hist():
    def clr(j, _):
      for s in range(_NSUB):
        hist[j, pl.ds(s * _L, _L)] = zi
      return 0
    lax.fori_loop(0, _NBINS, clr, 0)

  def stream_pass(row_fn, carry):
    def process(buf, carry):
      def row_body(i2, carry):
        for u in range(2):
          vecs = [buf[i2 * 2 + u, pl.ds(s * _L, _L)] for s in range(_NSUB)]
          carry = row_fn(vecs, carry)
        return carry
      return lax.fori_loop(0, _R // 2, row_body, carry)

    def pair_body(j, carry):
      pltpu.async_copy(slab(2 * j + 1), dbuf1, sem1)
      pltpu.make_async_copy(slab(0), dbuf0, sem0).wait()
      carry = process(dbuf0, carry)
      nxt = (2 * j + 2) % _NCHUNK
      pltpu.async_copy(slab(nxt), dbuf0, sem0)
      pltpu.make_async_copy(slab(0), dbuf1, sem1).wait()
      carry = process(dbuf1, carry)
      return carry
    return lax.fori_loop(0, _NCHUNK // 2, pair_body, carry)

  def scan_hist(s, need):
    def sc(j, carry):
      cum, p, above, found = carry
      bin_j = _NBINS - 1 - j
      cnt = hist[bin_j, pl.ds(s * _L, _L)]
      newcum = cum + cnt
      cross = jnp.logical_and(jnp.logical_not(found), newcum >= need)
      p = jnp.where(cross, bin_j, p)
      above = jnp.where(cross, cum, above)
      found = jnp.logical_or(found, cross)
      return newcum, p, above, found
    init = (zi, zi, zi, jnp.zeros((_L,), jnp.bool_))
    _, p, above, _ = lax.fori_loop(0, _NBINS, sc, init)
    return p, above

  pltpu.async_copy(slab(0), dbuf0, sem0)

  clear_hist()

  def hist1_row(vecs, carry):
    for s in range(_NSUB):
      key = _key_of(vecs[s])
      bn = lax.bitcast_convert_type(key >> jnp.uint32(24), jnp.int32)
      plsc.addupdate_scatter(hist, [bn, lanes + s * _L], ones)
    return carry
  stream_pass(hist1_row, 0)

  p1s, needs1, tots = [], [], []
  for s in range(_NSUB):
    p, above = scan_hist(s, jnp.full((_L,), _K, jnp.int32))
    pop = plsc.load_gather(hist, [p, lanes + s * _L])
    p1s.append(p)
    needs1.append(jnp.full((_L,), _K, jnp.int32) - above)
    tots.append(above + pop)

  ok = jnp.bool_(True)
  for s in range(_NSUB):
    ok = jnp.logical_and(ok, jnp.all(tots[s] <= _CAND))
    ok = jnp.logical_and(ok, jnp.all(p1s[s] >= 129))
  maxtot = tots[0]
  for s in range(1, _NSUB):
    maxtot = jnp.maximum(maxtot, tots[s])
  maxcc = jnp.max(maxtot)

  def fast_path():
    cutoffs = [lax.bitcast_convert_type((p1s[s] - 128) << jnp.int32(24),
                                        jnp.float32)
               for s in range(_NSUB)]

    def collect_row(vecs, carry):
      ccnts = list(carry)
      for s in range(_NSUB):
        m = vecs[s] >= cutoffs[s]
        plsc.store_scatter(clk, [ccnts[s], lanes + s * _L], vecs[s],
                           mask=m)
        ccnts[s] = ccnts[s] + jnp.where(m, 1, 0)
      return tuple(ccnts)
    stream_pass(collect_row, tuple([zi] * _NSUB))

    prefixes = [lax.bitcast_convert_type(p1s[s], jnp.uint32)
                for s in range(_NSUB)]
    needs = list(needs1)
    for shift, pshift in ((16, 24), (8, 16), (0, 8)):
      clear_hist()

      def cr(j, _, prefixes=prefixes, shift=shift, pshift=pshift):
        for s in range(_NSUB):
          kj = _key_of(clk[j, pl.ds(s * _L, _L)])
          m = jnp.logical_and(
              j < tots[s], (kj >> jnp.uint32(pshift)) == prefixes[s])
          bn = lax.bitcast_convert_type(
              (kj >> jnp.uint32(shift)) & jnp.uint32(0xFF), jnp.int32)
          plsc.addupdate_scatter(hist, [bn, lanes + s * _L], ones, mask=m)
        return 0
      lax.fori_loop(0, maxcc, cr, 0)

      for s in range(_NSUB):
        p, above = scan_hist(s, needs[s])
        prefixes[s] = (
            (prefixes[s] << jnp.uint32(8))
            | lax.bitcast_convert_type(p, jnp.uint32))
        needs[s] = needs[s] - above

    tvals = [_val_of(prefixes[s]) for s in range(_NSUB)]

    def emit(j, carry):
      outcnts, ties = list(carry[0]), list(carry[1])
      for s in range(_NSUB):
        vj = clk[j, pl.ds(s * _L, _L)]
        valid = j < tots[s]
        gt = vj > tvals[s]
        eq = jnp.logical_and(vj == tvals[s], ties[s] < needs[s])
        take = jnp.logical_and(valid, jnp.logical_or(gt, eq))
        plsc.store_scatter(outb, [outcnts[s], lanes + s * _L], vj,
                           mask=take)
        outcnts[s] = outcnts[s] + jnp.where(take, 1, 0)
        ties[s] = ties[s] + jnp.where(jnp.logical_and(take, eq), 1, 0)
      return tuple(outcnts), tuple(ties)
    lax.fori_loop(0, maxcc, emit, (tuple([zi] * _NSUB),
                                   tuple([zi] * _NSUB)))
    return 0

  def slow_path():
    prefixes = [lax.bitcast_convert_type(p1s[s], jnp.uint32)
                for s in range(_NSUB)]
    needs = list(needs1)
    for shift, pshift in ((16, 24), (8, 16), (0, 8)):
      clear_hist()

      def histk_row(vecs, carry, prefixes=prefixes, shift=shift,
                    pshift=pshift):
        for s in range(_NSUB):
          key = _key_of(vecs[s])
          m = (key >> jnp.uint32(pshift)) == prefixes[s]
          bn = lax.bitcast_convert_type(
              (key >> jnp.uint32(shift)) & jnp.uint32(0xFF), jnp.int32)
          plsc.addupdate_scatter(hist, [bn, lanes + s * _L], ones, mask=m)
        return carry
      stream_pass(histk_row, 0)

      for s in range(_NSUB):
        p, above = scan_hist(s, needs[s])
        prefixes[s] = (
            (prefixes[s] << jnp.uint32(8))
            | lax.bitcast_convert_type(p, jnp.uint32))
        needs[s] = needs[s] - above

    def sel_row(vecs, carry):
      outcnts, ties = list(carry[0]), list(carry[1])
      for s in range(_NSUB):
        key = _key_of(vecs[s])
        gt = key > prefixes[s]
        take_eq = jnp.logical_and(key == prefixes[s], ties[s] < needs[s])
        take = jnp.logical_or(gt, take_eq)
        plsc.store_scatter(outb, [outcnts[s], lanes + s * _L], vecs[s],
                           mask=take)
        outcnts[s] = outcnts[s] + jnp.where(take, 1, 0)
        ties[s] = ties[s] + jnp.where(take_eq, 1, 0)
      return tuple(outcnts), tuple(ties)
    stream_pass(sel_row, (tuple([zi] * _NSUB), tuple([zi] * _NSUB)))
    return 0

  lax.cond(ok, fast_path, slow_path)

  pltpu.make_async_copy(slab(0), dbuf0, sem0).wait()

  pltpu.sync_copy(outb, out_hbm.at[b, :, pl.ds(c0, _CB)])


@functools.partial(
    pl.kernel,
    out_type=jax.ShapeDtypeStruct((_B, _K, _C), jnp.float32),
    mesh=plsc.VectorSubcoreMesh(
        core_axis_name="c", subcore_axis_name="s",
        num_cores=_NC, num_subcores=_NS),
    scratch_types=[
        pltpu.VMEM((_R, _CB), jnp.float32),
        pltpu.VMEM((_R, _CB), jnp.float32),
        pltpu.VMEM((_NBINS, _CB), jnp.int32),
        pltpu.VMEM((_K, _CB), jnp.float32),
        pltpu.VMEM((_CAND, _CB), jnp.float32),
        pltpu.SemaphoreType.DMA,
        pltpu.SemaphoreType.DMA,
    ],
    compiler_params=pltpu.CompilerParams(needs_layout_passes=False),
)
def _kmax_sc(x_hbm, out_hbm, dbuf0, dbuf1, hist, outb, clk, sem0, sem1):
  _kmax_body(x_hbm, out_hbm, dbuf0, dbuf1, hist, outb, clk, sem0, sem1)


def kernel(x):
  return _kmax_sc(x)
